# Initial kernel scaffold; baseline (speedup 1.0000x reference)
import jax
import jax.numpy as jnp
from jax.experimental import pallas as pl


def kernel(x, edge_index, batch, W0, b0, W1, b1, W2, b2, W3, b3, conv1_w, conv1_b, conv2_w, conv2_b, lin1_w, lin1_b, lin2_w, lin2_b):
    raise NotImplementedError("write your pallas kernel here")


# speedup vs baseline: 3.4548x; 3.4548x over previous
import functools

import jax
import jax.numpy as jnp
import numpy as np
from jax import lax
from jax.experimental import pallas as pl
from jax.experimental.pallas import tpu as pltpu
from jax.experimental.pallas import tpu_sc as plsc

N = 10000
E = 320000
H = 32
B = 128
K = 64
DP = 112
NC = 2
NS = 16
NW = NC * NS
LN = 16
EW = E // NW
CH, CW = 125, 80
NSEG = N // NS
NPN = 313
NPADN = NW * NPN
SNP = 10048
WW = 2048
EP = E + WW
F32 = jnp.float32
I32 = jnp.int32

_mesh = plsc.VectorSubcoreMesh(core_axis_name="c", subcore_axis_name="s",
                               num_cores=NC, num_subcores=NS)
_sc_params = pltpu.CompilerParams(needs_layout_passes=False,
                                  use_tc_tiling_on_sc=False)


def _wid():
    return lax.axis_index("s") * NC + lax.axis_index("c")


NBW = 4
CB = B + LN


@functools.partial(
    pl.kernel, mesh=_mesh,
    out_type=(jax.ShapeDtypeStruct((NC, N, LN), F32),
              jax.ShapeDtypeStruct((NC, CB, LN), F32)),
    compiler_params=_sc_params,
    scratch_types=[pltpu.VMEM((CH, CW), I32), pltpu.VMEM((NBW, CW), I32),
                   pltpu.VMEM((CW, LN), F32), pltpu.VMEM((NSEG, LN), F32),
                   pltpu.VMEM_SHARED((N, LN), F32),
                   pltpu.VMEM_SHARED((CB, LN), F32)],
)
def _hist(dst_hbm, batch_hbm, deg_out, cnt_out, dstv, bv, onesr, zrows, dacc, cacc):
    c = lax.axis_index("c")
    s = lax.axis_index("s")
    w = _wid()
    ones = jnp.ones((LN,), F32)
    zeros = jnp.zeros((LN,), F32)

    def fill(i, _):
        onesr[i, pl.ds(0, LN)] = ones
        zrows[i, pl.ds(0, LN)] = zeros
        return 0
    lax.fori_loop(0, CW, fill, 0)

    def zb(i, _):
        zrows[i, pl.ds(0, LN)] = zeros
        return 0
    lax.fori_loop(CW, NSEG, zb, 0)

    pltpu.sync_copy(zrows, dacc.at[pl.ds(s * NSEG, NSEG)])

    @pl.when(s == 0)
    def _():
        pltpu.sync_copy(zrows.at[pl.ds(0, CB)], cacc)
    plsc.subcore_barrier()

    pltpu.sync_copy(dst_hbm.at[w], dstv)
    pltpu.sync_copy(batch_hbm.at[w], bv)

    def eb(i, _):
        pltpu.sync_copy(onesr, dacc.at[dstv.at[i]], add=True)
        return 0
    lax.fori_loop(0, CH, eb, 0)
    for i in range(NBW):
        pltpu.sync_copy(onesr, cacc.at[bv.at[i]], add=True)
    plsc.subcore_barrier()

    pltpu.sync_copy(dacc.at[pl.ds(s * NSEG, NSEG)],
                    deg_out.at[c, pl.ds(s * NSEG, NSEG)])

    @pl.when(s == 0)
    def _():
        pltpu.sync_copy(cacc, cnt_out.at[c])


@functools.partial(
    pl.kernel, mesh=_mesh,
    out_type=jax.ShapeDtypeStruct((SNP,), I32),
    compiler_params=_sc_params,
    scratch_types=[pltpu.VMEM((SNP,), F32), pltpu.VMEM((SNP,), I32)],
)
def _ncum(dege_hbm, out_hbm, dv, ov):
    w = _wid()

    @pl.when(w == 0)
    def _():
        pltpu.sync_copy(dege_hbm, dv)

        def body(i, run):
            v = dv[pl.ds(i * LN, LN)]
            cinc = plsc.cumsum(v)
            ov[pl.ds(i * LN, LN)] = (cinc - v + run).astype(I32)
            return run + jnp.sum(v)
        lax.fori_loop(0, SNP // LN, body, jnp.float32(0.0))
        pltpu.sync_copy(ov, out_hbm)


@functools.partial(
    pl.kernel, mesh=_mesh,
    out_type=jax.ShapeDtypeStruct((NW, N), F32),
    compiler_params=_sc_params,
    scratch_types=[pltpu.VMEM((EW,), I32), pltpu.VMEM((N,), F32)],
)
def _ehist(dstf_hbm, h_out, dstv, histv):
    w = _wid()
    zeros = jnp.zeros((LN,), F32)

    def zb(i, _):
        histv[pl.ds(i * LN, LN)] = zeros
        return 0
    lax.fori_loop(0, N // LN, zb, 0)
    pltpu.sync_copy(dstf_hbm.at[pl.ds(w * EW, EW)], dstv)

    def eb(i, _):
        v = dstv[pl.ds(i * LN, LN)]
        cnt, lastm = plsc.scan_count(v)
        plsc.addupdate_scatter(histv, [v], cnt.astype(F32), mask=lastm)
        return 0
    lax.fori_loop(0, EW // LN, eb, 0)
    pltpu.sync_copy(histv, h_out.at[w])


def _offs_body(h_ref, sn_ref, off_ref):
    tri = (lax.broadcasted_iota(I32, (NW, NW), 1)
           < lax.broadcasted_iota(I32, (NW, NW), 0)).astype(F32)
    excl = jnp.dot(tri, h_ref[...], preferred_element_type=F32)
    off_ref[...] = (excl + sn_ref[...].astype(F32)).astype(I32)


@functools.partial(
    pl.kernel, mesh=_mesh,
    out_type=(jax.ShapeDtypeStruct((E,), I32),
              jax.ShapeDtypeStruct((EP + LN,), I32),
              jax.ShapeDtypeStruct((EP,), F32)),
    compiler_params=_sc_params,
    scratch_types=[pltpu.VMEM((CH, CW), I32), pltpu.VMEM((CH, CW), I32),
                   pltpu.VMEM((CH, CW), I32), pltpu.VMEM((CH, CW), F32),
                   pltpu.VMEM((N,), I32), pltpu.VMEM((N,), I32),
                   pltpu.VMEM((N,), F32), pltpu.SemaphoreType.DMA],
)
def _esort(src_hbm, dst_hbm, off_hbm, dis_hbm, srcs_out, dsts_out, norm_out,
           srcb, dstb, posb, normb, offv, cntv, disv, sem):
    w = _wid()
    zeros_i = jnp.zeros((LN,), I32)

    def zb(i, _):
        cntv[pl.ds(i * LN, LN)] = zeros_i
        return 0
    lax.fori_loop(0, N // LN, zb, 0)

    pltpu.sync_copy(src_hbm.at[w], srcb)
    pltpu.sync_copy(dst_hbm.at[w], dstb)
    pltpu.sync_copy(off_hbm.at[w], offv)
    pltpu.sync_copy(dis_hbm, disv)

    def row(i, _):
        for j in range(CW // LN):
            v = dstb[i, pl.ds(j * LN, LN)]
            sv = srcb[i, pl.ds(j * LN, LN)]
            rank, lastm = plsc.scan_count(v)
            basec = plsc.load_gather(cntv, [v])
            offg = plsc.load_gather(offv, [v])
            posb[i, pl.ds(j * LN, LN)] = offg + basec + rank - 1
            plsc.addupdate_scatter(cntv, [v], rank, mask=lastm)
            da = plsc.load_gather(disv, [sv])
            db = plsc.load_gather(disv, [v])
            normb[i, pl.ds(j * LN, LN)] = da * db
        d1 = pltpu.async_copy(srcb.at[i], srcs_out.at[posb.at[i]], sem)
        d2 = pltpu.async_copy(dstb.at[i], dsts_out.at[posb.at[i]], sem)
        d3 = pltpu.async_copy(normb.at[i], norm_out.at[posb.at[i]], sem)
        d1.wait(); d2.wait(); d3.wait()
        return 0
    lax.fori_loop(0, CH, row, 0)


def _make_egather(hc):
    @functools.partial(
        pl.kernel, mesh=_mesh,
        out_type=jax.ShapeDtypeStruct((EP, hc), F32),
        compiler_params=_sc_params,
        scratch_types=[pltpu.VMEM((CH, CW), I32), pltpu.VMEM((CW, hc), F32),
                       pltpu.SemaphoreType.DMA],
    )
    def egather(p_hbm, srcs_hbm, pg_out, srcb, rows, sem):
        w = _wid()
        pltpu.sync_copy(srcs_hbm.at[w], srcb)

        def row(i, _):
            pltpu.async_copy(p_hbm.at[srcb.at[i]], rows, sem).wait()
            pltpu.sync_copy(rows, pg_out.at[pl.ds(w * EW + i * CW, CW)])
            return 0
        lax.fori_loop(0, CH, row, 0)
    return egather


_egather32 = _make_egather(H)
_egather16 = _make_egather(LN)


def _emul_body(pg_ref, norm_ref, msg_ref):
    msg_ref[...] = pg_ref[...] * norm_ref[...]


def _make_emul(hc, grid=32):
    te = EP // grid
    return pl.pallas_call(
        _emul_body,
        grid=(grid,),
        in_specs=[pl.BlockSpec((te, hc), lambda i: (i, 0)),
                  pl.BlockSpec((te, 1), lambda i: (i, 0))],
        out_specs=pl.BlockSpec((te, hc), lambda i: (i, 0)),
        out_shape=jax.ShapeDtypeStruct((EP, hc), F32),
    )


_emul32 = _make_emul(H)
_emul16 = _make_emul(LN)


def _make_ereduce(hc):
    nv = hc // LN

    @functools.partial(
        pl.kernel, mesh=_mesh,
        out_type=jax.ShapeDtypeStruct((NPADN * hc,), F32),
        compiler_params=_sc_params,
        scratch_types=[pltpu.VMEM((SNP + LN,), I32),
                       pltpu.VMEM((NPN * hc,), F32),
                       pltpu.VMEM((WW * hc,), F32),
                       pltpu.VMEM((WW + LN,), I32)],
    )
    def ereduce(msgf_hbm, dsts_hbm, sn_hbm, a_out, snv, accb, ebuf, dbuf):
        w = _wid()
        zeros = jnp.zeros((LN,), F32)

        pltpu.sync_copy(sn_hbm, snv.at[pl.ds(0, SNP)])
        snv[pl.ds(SNP, LN)] = jnp.full((LN,), E, I32)

        def zb(i, _):
            accb[pl.ds(i * LN, LN)] = zeros
            return 0
        lax.fori_loop(0, NPN * hc // LN, zb, 0)

        lo = snv[pl.ds(w * NPN, LN)][0]
        hi = snv[pl.ds((w + 1) * NPN, LN)][0]
        base = (lo // 8) * 8
        nodebase = w * NPN
        nwin = (hi - base + WW - 1) // WW

        def win(t, carry):
            wstart = base + t * WW
            pltpu.sync_copy(msgf_hbm.at[pl.ds(wstart * hc, WW * hc)], ebuf)
            pltpu.sync_copy(dsts_hbm.at[pl.ds(wstart, WW + LN)], dbuf)
            estart = jnp.maximum(lo - wstart, 0)
            eend = jnp.minimum(hi - wstart, WW)

            def edge(e, ec):
                prev = ec[0]
                row = dbuf[pl.ds(e, LN)][0] - nodebase
                same = row == prev
                accs = []
                for q in range(nv):
                    m = ebuf[pl.ds(e * hc + q * LN, LN)]
                    a = jnp.where(same, ec[1 + q] + m, m)
                    accb[pl.ds(row * hc + q * LN, LN)] = a
                    accs.append(a)
                return (row, *accs)

            return lax.fori_loop(estart, eend, edge, carry)

        init = (jnp.int32(-1),) + tuple(zeros for _ in range(nv))
        lax.fori_loop(0, nwin, win, init)
        pltpu.sync_copy(accb, a_out.at[pl.ds(w * NPN * hc, NPN * hc)])
    return ereduce


_ereduce32 = _make_ereduce(H)
_ereduce16 = _make_ereduce(LN)


GPW = B // NW


@functools.partial(
    pl.kernel, mesh=_mesh,
    out_type=jax.ShapeDtypeStruct((B * K, DP), F32),
    compiler_params=_sc_params,
    scratch_types=[pltpu.VMEM((N,), F32), pltpu.VMEM((GPW, K), I32),
                   pltpu.VMEM((GPW * K, DP), F32),
                   pltpu.VMEM((B,), I32), pltpu.VMEM((B,), I32),
                   pltpu.SemaphoreType.DMA],
)
def _select(key_hbm, starts_hbm, counts_hbm, xcat_hbm, out_hbm,
            keyv, selv, rows, sv, cv, sem):
    w = _wid()
    iota = lax.iota(I32, LN)
    inf = jnp.float32(np.inf)
    lane0 = iota == 0

    pltpu.sync_copy(key_hbm, keyv)
    pltpu.sync_copy(starts_hbm, sv)
    pltpu.sync_copy(counts_hbm, cv)

    for kk in range(GPW):
        for q in range(K // LN):
            selv[kk, pl.ds(q * LN, LN)] = N + iota

    for kk in range(GPW):
        g = w * GPW + kk
        gbase = (g // LN) * LN
        lane = g - gbase
        sv16 = sv[pl.ds(gbase, LN)]
        cv16 = cv[pl.ds(gbase, LN)]
        start = jnp.max(jnp.where(iota == lane, sv16, -1))
        cnt = jnp.max(jnp.where(iota == lane, cv16, -1))
        nch = (cnt + LN - 1) // LN
        nrounds = jnp.minimum(cnt, K)

        def rb(r, _):
            def cb(j, carry):
                bv_, bi_ = carry
                pos = j * LN + iota
                idx = start + pos
                m = pos < cnt
                v = plsc.load_gather(keyv, [idx], mask=m)
                v = jnp.where(m, v, inf)
                upd = v < bv_
                return (jnp.where(upd, v, bv_), jnp.where(upd, idx, bi_))

            bv_, bi_ = lax.fori_loop(
                0, nch, cb,
                (jnp.full((LN,), inf, F32), jnp.zeros((LN,), I32)))
            mn = jnp.min(bv_)
            selidx = jnp.min(jnp.where(bv_ == mn, bi_, jnp.int32(1 << 30)))
            plsc.store_scatter(
                selv,
                [jnp.full((LN,), kk, I32), jnp.full((LN,), r, I32)],
                jnp.full((LN,), selidx, I32), mask=lane0)
            plsc.store_scatter(keyv, [jnp.full((LN,), selidx, I32)],
                               jnp.full((LN,), inf, F32), mask=lane0)
            return 0
        lax.fori_loop(0, nrounds, rb, 0)

    for kk in range(GPW):
        pltpu.async_copy(xcat_hbm.at[selv.at[kk]],
                         rows.at[pl.ds(kk * K, K)], sem).wait()
    pltpu.sync_copy(rows, out_hbm.at[pl.ds(w * GPW * K, GPW * K)])


def _tc_call(body, out_shapes):
    return pl.pallas_call(body, out_shape=out_shapes)


def _prep_body(degp_ref, cntp_ref, x_ref, w0_ref,
               p1_ref, dis_ref, disq_ref, dege_ref, counts_ref, starts_ref):
    dp = degp_ref[...]
    dege = (dp[0] + dp[1])[:, 0:1]
    dege_ref[...] = dege
    dis = lax.rsqrt(dege + 1.0)
    dis_ref[...] = dis
    disq_ref[...] = dis * dis
    p1_ref[...] = jnp.dot(x_ref[...], w0_ref[...],
                          preferred_element_type=F32)
    cp = cntp_ref[...]
    ccol = (cp[0] + cp[1])[0:B, 0:1]
    counts_ref[...] = ccol.astype(I32)
    tri = (lax.broadcasted_iota(I32, (B, B), 1)
           < lax.broadcasted_iota(I32, (B, B), 0)).astype(F32)
    starts_ref[...] = jnp.dot(tri, ccol,
                              preferred_element_type=F32).astype(I32)


def _mid_body(a_ref, p_ref, disq_ref, b_ref, wn_ref, h_ref, pn_ref,
              *, pad_out):
    a = a_ref[...][0:N]
    t = a + p_ref[...] * disq_ref[...]
    h = jnp.tanh(t + b_ref[...])
    h_ref[...] = h
    pn = jnp.dot(h, wn_ref[...], preferred_element_type=F32)
    if pad_out:
        pn = jnp.concatenate(
            [pn, jnp.zeros((N, LN - pn.shape[1]), F32)], axis=1)
    pn_ref[...] = pn


def _final_body(a_ref, p_ref, disq_ref, b_ref, batchf_ref, h4_ref, key_ref):
    a = a_ref[...][0:N, 0:1]
    t = a + p_ref[...][:, 0:1] * disq_ref[...]
    h4 = jnp.tanh(t + b_ref[...])
    h4_ref[...] = h4
    key_ref[...] = batchf_ref[...] * 4.0 - h4


def _tail1_body(pooled2_ref, c1w2_ref, c1b2_ref, zp_ref):
    z1 = jnp.dot(pooled2_ref[...], c1w2_ref[...],
                 preferred_element_type=F32) + c1b2_ref[...]
    z1 = jnp.maximum(z1, 0.0)
    zp_ref[...] = jnp.maximum(z1[:, :16], z1[:, 16:])


def _tail2_body(g_ref, w2big_ref, c2b_ref, l1w_ref, l1b_ref, l2w_ref,
                l2b_ref, out_ref):
    z2 = jnp.dot(g_ref[...], w2big_ref[...],
                 preferred_element_type=F32) + c2b_ref[...]
    z2 = jnp.maximum(z2, 0.0)
    z = jnp.maximum(jnp.dot(z2, l1w_ref[...],
                            preferred_element_type=F32) + l1b_ref[...], 0.0)
    out_ref[...] = jnp.dot(z, l2w_ref[...],
                           preferred_element_type=F32) + l2b_ref[...]


_prep = _tc_call(_prep_body,
                 (jax.ShapeDtypeStruct((N, H), F32),
                  jax.ShapeDtypeStruct((N, 1), F32),
                  jax.ShapeDtypeStruct((N, 1), F32),
                  jax.ShapeDtypeStruct((N, 1), F32),
                  jax.ShapeDtypeStruct((B, 1), I32),
                  jax.ShapeDtypeStruct((B, 1), I32)))
_offs = _tc_call(_offs_body, jax.ShapeDtypeStruct((NW, N), I32))
_mid = _tc_call(functools.partial(_mid_body, pad_out=False),
                (jax.ShapeDtypeStruct((N, H), F32),
                 jax.ShapeDtypeStruct((N, H), F32)))
_mid_pad = _tc_call(functools.partial(_mid_body, pad_out=True),
                    (jax.ShapeDtypeStruct((N, H), F32),
                     jax.ShapeDtypeStruct((N, LN), F32)))
_final = _tc_call(_final_body,
                  (jax.ShapeDtypeStruct((N, 1), F32),
                   jax.ShapeDtypeStruct((N, 1), F32)))
_tail1 = _tc_call(_tail1_body, jax.ShapeDtypeStruct((B * K // 2, LN), F32))
_tail2 = _tc_call(_tail2_body, jax.ShapeDtypeStruct((B, 1), F32))


def kernel(x, edge_index, batch, W0, b0, W1, b1, W2, b2, W3, b3,
           conv1_w, conv1_b, conv2_w, conv2_b,
           lin1_w, lin1_b, lin2_w, lin2_b):
    src3 = edge_index[0].reshape(NW, CH, CW)
    dst3 = edge_index[1].reshape(NW, CH, CW)
    dstf = edge_index[1]
    batch_pad = jnp.concatenate(
        [batch, jnp.full((NW * NBW * CW - N,), B, I32)]).reshape(NW, NBW, CW)

    degp, cntp = _hist(dst3, batch_pad)
    p1, dis, disq, dege, counts, starts = _prep(degp, cntp, x, W0)

    dege_pad = jnp.concatenate([dege.reshape(N), jnp.zeros((SNP - N,), F32)])
    startsN = _ncum(dege_pad)

    hw = _ehist(dstf)
    off = _offs(hw, startsN[:N].reshape(1, N))
    srcS, dstS, normS = _esort(src3, dst3, off, dis.reshape(N))
    srcS3 = srcS.reshape(NW, CH, CW)
    norm2 = normS.reshape(EP, 1)

    def mp(p, hc):
        eg = _egather32 if hc == H else _egather16
        em = _emul32 if hc == H else _emul16
        er = _ereduce32 if hc == H else _ereduce16
        pg = eg(p, srcS3)
        msg = em(pg, norm2)
        a = er(msg.reshape(EP * hc), dstS, startsN)
        return a.reshape(NPADN, hc)

    a1 = mp(p1, H)
    h1, p2 = _mid(a1, p1, disq, b0.reshape(1, H), W1)
    a2 = mp(p2, H)
    h2, p3 = _mid(a2, p2, disq, b1.reshape(1, H), W2)
    a3 = mp(p3, H)
    h3, p4 = _mid_pad(a3, p3, disq, b2.reshape(1, H), W3)
    a4 = mp(p4, LN)
    batchf = batch.astype(F32)[:, None]
    h4, key = _final(a4, p4, disq, b3.reshape(1, 1), batchf)

    xcat = jnp.concatenate(
        [h1, h2, h3, h4, jnp.zeros((N, DP - 97), F32)], axis=1)
    xcat = jnp.concatenate([xcat, jnp.zeros((LN, DP), F32)], axis=0)

    pooled = _select(key.reshape(N), starts.reshape(B), counts.reshape(B),
                     xcat)

    c1w = jnp.pad(conv1_w.reshape(16, 97).T, ((0, DP - 97), (0, 0)))
    c1w2 = jnp.kron(jnp.eye(2, dtype=F32), c1w)
    c1b2 = jnp.tile(conv1_b, 2).reshape(1, 32)
    zp = _tail1(pooled.reshape(B * K // 2, 2 * DP), c1w2, c1b2)

    w2big = sum(
        jnp.pad(jnp.kron(jnp.eye(28, dtype=F32), conv2_w[:, :, j].T),
                ((16 * j, 64 - 16 * j), (0, 0)))
        for j in range(5))
    b2big = jnp.tile(conv2_b, 28).reshape(1, 896)
    lin1p = lin1_w.reshape(32, 28, 128).transpose(1, 0, 2).reshape(896, 128)
    out = _tail2(zp.reshape(B, (K // 2) * LN), w2big, b2big,
                 lin1p, lin1_b.reshape(1, 128), lin2_w, lin2_b.reshape(1, 1))
    return out


# speedup vs baseline: 3.5899x; 1.0391x over previous
import functools

import jax
import jax.numpy as jnp
import numpy as np
from jax import lax
from jax.experimental import pallas as pl
from jax.experimental.pallas import tpu as pltpu
from jax.experimental.pallas import tpu_sc as plsc

N = 10000
E = 320000
H = 32
B = 128
K = 64
DP = 112
NC = 2
NS = 16
NW = NC * NS
LN = 16
EW = E // NW
CH, CW = 125, 80
NSEG = N // NS
NPN = 313
NPADN = NW * NPN
SNP = 10048
WW = 2048
EP = E + WW
F32 = jnp.float32
I32 = jnp.int32

_mesh = plsc.VectorSubcoreMesh(core_axis_name="c", subcore_axis_name="s",
                               num_cores=NC, num_subcores=NS)
_sc_params = pltpu.CompilerParams(needs_layout_passes=False,
                                  use_tc_tiling_on_sc=False)


def _wid():
    return lax.axis_index("s") * NC + lax.axis_index("c")


NBW = 4
CB = B + LN


@functools.partial(
    pl.kernel, mesh=_mesh,
    out_type=(jax.ShapeDtypeStruct((NC, N, LN), F32),
              jax.ShapeDtypeStruct((NC, CB, LN), F32)),
    compiler_params=_sc_params,
    scratch_types=[pltpu.VMEM((CH, CW), I32), pltpu.VMEM((NBW, CW), I32),
                   pltpu.VMEM((CW, LN), F32), pltpu.VMEM((NSEG, LN), F32),
                   pltpu.VMEM_SHARED((N, LN), F32),
                   pltpu.VMEM_SHARED((CB, LN), F32),
                   pltpu.SemaphoreType.DMA],
)
def _hist(dst_hbm, batch_hbm, deg_out, cnt_out, dstv, bv, onesr, zrows, dacc, cacc, dsem):
    c = lax.axis_index("c")
    s = lax.axis_index("s")
    w = _wid()
    ones = jnp.ones((LN,), F32)
    zeros = jnp.zeros((LN,), F32)

    def fill(i, _):
        onesr[i, pl.ds(0, LN)] = ones
        zrows[i, pl.ds(0, LN)] = zeros
        return 0
    lax.fori_loop(0, CW, fill, 0)

    def zb(i, _):
        zrows[i, pl.ds(0, LN)] = zeros
        return 0
    lax.fori_loop(CW, NSEG, zb, 0)

    pltpu.sync_copy(zrows, dacc.at[pl.ds(s * NSEG, NSEG)])

    @pl.when(s == 0)
    def _():
        pltpu.sync_copy(zrows.at[pl.ds(0, CB)], cacc)
    plsc.subcore_barrier()

    pltpu.sync_copy(dst_hbm.at[w], dstv)
    pltpu.sync_copy(batch_hbm.at[w], bv)

    def eb(i, _):
        pltpu.async_copy(onesr, dacc.at[dstv.at[i]], dsem, add=True)

        @pl.when(i >= 8)
        def _():
            pltpu.make_async_copy(onesr, dacc.at[dstv.at[i - 8]], dsem).wait()
        return 0
    lax.fori_loop(0, CH, eb, 0)

    def ebd(i, _):
        pltpu.make_async_copy(onesr, dacc.at[dstv.at[CH - 8 + i]], dsem).wait()
        return 0
    lax.fori_loop(0, 8, ebd, 0)
    for i in range(NBW):
        pltpu.sync_copy(onesr, cacc.at[bv.at[i]], add=True)
    plsc.subcore_barrier()

    pltpu.sync_copy(dacc.at[pl.ds(s * NSEG, NSEG)],
                    deg_out.at[c, pl.ds(s * NSEG, NSEG)])

    @pl.when(s == 0)
    def _():
        pltpu.sync_copy(cacc, cnt_out.at[c])


@functools.partial(
    pl.kernel, mesh=_mesh,
    out_type=jax.ShapeDtypeStruct((SNP,), I32),
    compiler_params=_sc_params,
    scratch_types=[pltpu.VMEM((SNP,), F32), pltpu.VMEM((SNP,), I32)],
)
def _ncum(dege_hbm, out_hbm, dv, ov):
    w = _wid()

    @pl.when(w == 0)
    def _():
        pltpu.sync_copy(dege_hbm, dv)

        def body(i, run):
            v = dv[pl.ds(i * LN, LN)]
            cinc = plsc.cumsum(v)
            ov[pl.ds(i * LN, LN)] = (cinc - v + run).astype(I32)
            return run + jnp.sum(v)
        lax.fori_loop(0, SNP // LN, body, jnp.float32(0.0))
        pltpu.sync_copy(ov, out_hbm)


@functools.partial(
    pl.kernel, mesh=_mesh,
    out_type=jax.ShapeDtypeStruct((NW, N), F32),
    compiler_params=_sc_params,
    scratch_types=[pltpu.VMEM((EW,), I32), pltpu.VMEM((N,), F32)],
)
def _ehist(dstf_hbm, h_out, dstv, histv):
    w = _wid()
    zeros = jnp.zeros((LN,), F32)

    def zb(i, _):
        histv[pl.ds(i * LN, LN)] = zeros
        return 0
    lax.fori_loop(0, N // LN, zb, 0)
    pltpu.sync_copy(dstf_hbm.at[pl.ds(w * EW, EW)], dstv)

    def eb(i, _):
        v = dstv[pl.ds(i * LN, LN)]
        cnt, lastm = plsc.scan_count(v)
        plsc.addupdate_scatter(histv, [v], cnt.astype(F32), mask=lastm)
        return 0
    lax.fori_loop(0, EW // LN, eb, 0)
    pltpu.sync_copy(histv, h_out.at[w])


def _offs_body(h_ref, sn_ref, off_ref):
    tri = (lax.broadcasted_iota(I32, (NW, NW), 1)
           < lax.broadcasted_iota(I32, (NW, NW), 0)).astype(F32)
    excl = jnp.dot(tri, h_ref[...], preferred_element_type=F32)
    off_ref[...] = (excl + sn_ref[...].astype(F32)).astype(I32)


@functools.partial(
    pl.kernel, mesh=_mesh,
    out_type=(jax.ShapeDtypeStruct((E,), I32),
              jax.ShapeDtypeStruct((EP + LN,), I32),
              jax.ShapeDtypeStruct((EP,), F32)),
    compiler_params=_sc_params,
    scratch_types=[pltpu.VMEM((CH, CW), I32), pltpu.VMEM((CH, CW), I32),
                   pltpu.VMEM((CH, CW), I32), pltpu.VMEM((CH, CW), F32),
                   pltpu.VMEM((N,), I32), pltpu.VMEM((N,), I32),
                   pltpu.VMEM((N,), F32), pltpu.SemaphoreType.DMA],
)
def _esort(src_hbm, dst_hbm, off_hbm, dis_hbm, srcs_out, dsts_out, norm_out,
           srcb, dstb, posb, normb, offv, cntv, disv, sem):
    w = _wid()
    zeros_i = jnp.zeros((LN,), I32)

    def zb(i, _):
        cntv[pl.ds(i * LN, LN)] = zeros_i
        return 0
    lax.fori_loop(0, N // LN, zb, 0)

    pltpu.sync_copy(src_hbm.at[w], srcb)
    pltpu.sync_copy(dst_hbm.at[w], dstb)
    pltpu.sync_copy(off_hbm.at[w], offv)
    pltpu.sync_copy(dis_hbm, disv)

    LAG = 8

    def _drain(j):
        pltpu.make_async_copy(srcb.at[j], srcs_out.at[posb.at[j]], sem).wait()
        pltpu.make_async_copy(dstb.at[j], dsts_out.at[posb.at[j]], sem).wait()
        pltpu.make_async_copy(normb.at[j], norm_out.at[posb.at[j]], sem).wait()

    def row(i, _):
        for j in range(CW // LN):
            v = dstb[i, pl.ds(j * LN, LN)]
            sv = srcb[i, pl.ds(j * LN, LN)]
            rank, lastm = plsc.scan_count(v)
            basec = plsc.load_gather(cntv, [v])
            offg = plsc.load_gather(offv, [v])
            posb[i, pl.ds(j * LN, LN)] = offg + basec + rank - 1
            plsc.addupdate_scatter(cntv, [v], rank, mask=lastm)
            da = plsc.load_gather(disv, [sv])
            db = plsc.load_gather(disv, [v])
            normb[i, pl.ds(j * LN, LN)] = da * db
        pltpu.async_copy(srcb.at[i], srcs_out.at[posb.at[i]], sem)
        pltpu.async_copy(dstb.at[i], dsts_out.at[posb.at[i]], sem)
        pltpu.async_copy(normb.at[i], norm_out.at[posb.at[i]], sem)

        @pl.when(i >= LAG)
        def _():
            _drain(i - LAG)
        return 0
    lax.fori_loop(0, CH, row, 0)

    def drain_tail(j, _):
        _drain(CH - LAG + j)
        return 0
    lax.fori_loop(0, LAG, drain_tail, 0)


def _make_egather(hc):
    @functools.partial(
        pl.kernel, mesh=_mesh,
        out_type=jax.ShapeDtypeStruct((EP, hc), F32),
        compiler_params=_sc_params,
        scratch_types=[pltpu.VMEM((CH, CW), I32), pltpu.VMEM((2 * CW, hc), F32),
                       pltpu.SemaphoreType.DMA],
    )
    def egather(p_hbm, srcs_hbm, pg_out, srcb, rows, sem):
        w = _wid()
        pltpu.sync_copy(srcs_hbm.at[w], srcb)
        pltpu.async_copy(p_hbm.at[srcb.at[0]], rows.at[pl.ds(0, CW)], sem)

        def row(i, _):
            @pl.when(i + 1 < CH)
            def _():
                pltpu.async_copy(p_hbm.at[srcb.at[i + 1]],
                                 rows.at[pl.ds(((i + 1) % 2) * CW, CW)], sem)
            cur = (i % 2) * CW
            pltpu.make_async_copy(p_hbm.at[srcb.at[i]],
                                  rows.at[pl.ds(cur, CW)], sem).wait()
            pltpu.sync_copy(rows.at[pl.ds(cur, CW)],
                            pg_out.at[pl.ds(w * EW + i * CW, CW)])
            return 0
        lax.fori_loop(0, CH, row, 0)
    return egather


_egather32 = _make_egather(H)
_egather16 = _make_egather(LN)


def _emul_body(pg_ref, norm_ref, msg_ref):
    msg_ref[...] = pg_ref[...] * norm_ref[...]


def _make_emul(hc, grid=32):
    te = EP // grid
    return pl.pallas_call(
        _emul_body,
        grid=(grid,),
        in_specs=[pl.BlockSpec((te, hc), lambda i: (i, 0)),
                  pl.BlockSpec((te, 1), lambda i: (i, 0))],
        out_specs=pl.BlockSpec((te, hc), lambda i: (i, 0)),
        out_shape=jax.ShapeDtypeStruct((EP, hc), F32),
    )


_emul32 = _make_emul(H)
_emul16 = _make_emul(LN)


def _make_ereduce(hc):
    nv = hc // LN

    @functools.partial(
        pl.kernel, mesh=_mesh,
        out_type=jax.ShapeDtypeStruct((NPADN * hc,), F32),
        compiler_params=_sc_params,
        scratch_types=[pltpu.VMEM((SNP + LN,), I32),
                       pltpu.VMEM((NPN * hc,), F32),
                       pltpu.VMEM((WW * hc,), F32),
                       pltpu.VMEM((WW + LN,), I32)],
    )
    def ereduce(msgf_hbm, dsts_hbm, sn_hbm, a_out, snv, accb, ebuf, dbuf):
        w = _wid()
        zeros = jnp.zeros((LN,), F32)

        pltpu.sync_copy(sn_hbm, snv.at[pl.ds(0, SNP)])
        snv[pl.ds(SNP, LN)] = jnp.full((LN,), E, I32)

        def zb(i, _):
            accb[pl.ds(i * LN, LN)] = zeros
            return 0
        lax.fori_loop(0, NPN * hc // LN, zb, 0)

        lo = snv[pl.ds(w * NPN, LN)][0]
        hi = snv[pl.ds((w + 1) * NPN, LN)][0]
        base = (lo // 8) * 8
        nodebase = w * NPN
        nwin = (hi - base + WW - 1) // WW

        def win(t, carry):
            wstart = base + t * WW
            pltpu.sync_copy(msgf_hbm.at[pl.ds(wstart * hc, WW * hc)], ebuf)
            pltpu.sync_copy(dsts_hbm.at[pl.ds(wstart, WW + LN)], dbuf)
            estart = jnp.maximum(lo - wstart, 0)
            eend = jnp.minimum(hi - wstart, WW)

            def edge(e, ec):
                prev = ec[0]
                row = dbuf[pl.ds(e, LN)][0] - nodebase
                same = row == prev
                accs = []
                for q in range(nv):
                    m = ebuf[pl.ds(e * hc + q * LN, LN)]
                    a = jnp.where(same, ec[1 + q] + m, m)
                    accb[pl.ds(row * hc + q * LN, LN)] = a
                    accs.append(a)
                return (row, *accs)

            return lax.fori_loop(estart, eend, edge, carry)

        init = (jnp.int32(-1),) + tuple(zeros for _ in range(nv))
        lax.fori_loop(0, nwin, win, init)
        pltpu.sync_copy(accb, a_out.at[pl.ds(w * NPN * hc, NPN * hc)])
    return ereduce


_ereduce32 = _make_ereduce(H)
_ereduce16 = _make_ereduce(LN)


GPW = B // NW


@functools.partial(
    pl.kernel, mesh=_mesh,
    out_type=jax.ShapeDtypeStruct((B * K, DP), F32),
    compiler_params=_sc_params,
    scratch_types=[pltpu.VMEM((N,), F32), pltpu.VMEM((GPW, K), I32),
                   pltpu.VMEM((GPW * K, DP), F32),
                   pltpu.VMEM((B,), I32), pltpu.VMEM((B,), I32),
                   pltpu.SemaphoreType.DMA],
)
def _select(key_hbm, starts_hbm, counts_hbm, xcat_hbm, out_hbm,
            keyv, selv, rows, sv, cv, sem):
    w = _wid()
    iota = lax.iota(I32, LN)
    inf = jnp.float32(np.inf)
    lane0 = iota == 0

    pltpu.sync_copy(key_hbm, keyv)
    pltpu.sync_copy(starts_hbm, sv)
    pltpu.sync_copy(counts_hbm, cv)

    for kk in range(GPW):
        for q in range(K // LN):
            selv[kk, pl.ds(q * LN, LN)] = N + iota

    for kk in range(GPW):
        g = w * GPW + kk
        gbase = (g // LN) * LN
        lane = g - gbase
        sv16 = sv[pl.ds(gbase, LN)]
        cv16 = cv[pl.ds(gbase, LN)]
        start = jnp.max(jnp.where(iota == lane, sv16, -1))
        cnt = jnp.max(jnp.where(iota == lane, cv16, -1))
        nch = (cnt + LN - 1) // LN
        nrounds = jnp.minimum(cnt, K)

        def rb(r, _):
            def cb(j, carry):
                bv_, bi_ = carry
                pos = j * LN + iota
                idx = start + pos
                m = pos < cnt
                v = plsc.load_gather(keyv, [idx], mask=m)
                v = jnp.where(m, v, inf)
                upd = v < bv_
                return (jnp.where(upd, v, bv_), jnp.where(upd, idx, bi_))

            bv_, bi_ = lax.fori_loop(
                0, nch, cb,
                (jnp.full((LN,), inf, F32), jnp.zeros((LN,), I32)))
            mn = jnp.min(bv_)
            selidx = jnp.min(jnp.where(bv_ == mn, bi_, jnp.int32(1 << 30)))
            plsc.store_scatter(
                selv,
                [jnp.full((LN,), kk, I32), jnp.full((LN,), r, I32)],
                jnp.full((LN,), selidx, I32), mask=lane0)
            plsc.store_scatter(keyv, [jnp.full((LN,), selidx, I32)],
                               jnp.full((LN,), inf, F32), mask=lane0)
            return 0
        lax.fori_loop(0, nrounds, rb, 0)

    for kk in range(GPW):
        pltpu.async_copy(xcat_hbm.at[selv.at[kk]],
                         rows.at[pl.ds(kk * K, K)], sem).wait()
    pltpu.sync_copy(rows, out_hbm.at[pl.ds(w * GPW * K, GPW * K)])


def _tc_call(body, out_shapes):
    return pl.pallas_call(body, out_shape=out_shapes)


def _prep_body(degp_ref, cntp_ref, x_ref, w0_ref,
               p1_ref, dis_ref, disq_ref, dege_ref, counts_ref, starts_ref):
    dp = degp_ref[...]
    dege = (dp[0] + dp[1])[:, 0:1]
    dege_ref[...] = dege
    dis = lax.rsqrt(dege + 1.0)
    dis_ref[...] = dis
    disq_ref[...] = dis * dis
    p1_ref[...] = jnp.dot(x_ref[...], w0_ref[...],
                          preferred_element_type=F32)
    cp = cntp_ref[...]
    ccol = (cp[0] + cp[1])[0:B, 0:1]
    counts_ref[...] = ccol.astype(I32)
    tri = (lax.broadcasted_iota(I32, (B, B), 1)
           < lax.broadcasted_iota(I32, (B, B), 0)).astype(F32)
    starts_ref[...] = jnp.dot(tri, ccol,
                              preferred_element_type=F32).astype(I32)


def _mid_body(a_ref, p_ref, disq_ref, b_ref, wn_ref, h_ref, pn_ref,
              *, pad_out):
    a = a_ref[...][0:N]
    t = a + p_ref[...] * disq_ref[...]
    h = jnp.tanh(t + b_ref[...])
    h_ref[...] = h
    pn = jnp.dot(h, wn_ref[...], preferred_element_type=F32)
    if pad_out:
        pn = jnp.concatenate(
            [pn, jnp.zeros((N, LN - pn.shape[1]), F32)], axis=1)
    pn_ref[...] = pn


def _final_body(a_ref, p_ref, disq_ref, b_ref, batchf_ref, h4_ref, key_ref):
    a = a_ref[...][0:N, 0:1]
    t = a + p_ref[...][:, 0:1] * disq_ref[...]
    h4 = jnp.tanh(t + b_ref[...])
    h4_ref[...] = h4
    key_ref[...] = batchf_ref[...] * 4.0 - h4


def _tail1_body(pooled2_ref, c1w2_ref, c1b2_ref, zp_ref):
    z1 = jnp.dot(pooled2_ref[...], c1w2_ref[...],
                 preferred_element_type=F32) + c1b2_ref[...]
    z1 = jnp.maximum(z1, 0.0)
    zp_ref[...] = jnp.maximum(z1[:, :16], z1[:, 16:])


def _tail2_body(g_ref, w2big_ref, c2b_ref, l1w_ref, l1b_ref, l2w_ref,
                l2b_ref, out_ref):
    z2 = jnp.dot(g_ref[...], w2big_ref[...],
                 preferred_element_type=F32) + c2b_ref[...]
    z2 = jnp.maximum(z2, 0.0)
    z = jnp.maximum(jnp.dot(z2, l1w_ref[...],
                            preferred_element_type=F32) + l1b_ref[...], 0.0)
    out_ref[...] = jnp.dot(z, l2w_ref[...],
                           preferred_element_type=F32) + l2b_ref[...]


_prep = _tc_call(_prep_body,
                 (jax.ShapeDtypeStruct((N, H), F32),
                  jax.ShapeDtypeStruct((N, 1), F32),
                  jax.ShapeDtypeStruct((N, 1), F32),
                  jax.ShapeDtypeStruct((N, 1), F32),
                  jax.ShapeDtypeStruct((B, 1), I32),
                  jax.ShapeDtypeStruct((B, 1), I32)))
_offs = _tc_call(_offs_body, jax.ShapeDtypeStruct((NW, N), I32))
_mid = _tc_call(functools.partial(_mid_body, pad_out=False),
                (jax.ShapeDtypeStruct((N, H), F32),
                 jax.ShapeDtypeStruct((N, H), F32)))
_mid_pad = _tc_call(functools.partial(_mid_body, pad_out=True),
                    (jax.ShapeDtypeStruct((N, H), F32),
                     jax.ShapeDtypeStruct((N, LN), F32)))
_final = _tc_call(_final_body,
                  (jax.ShapeDtypeStruct((N, 1), F32),
                   jax.ShapeDtypeStruct((N, 1), F32)))
_tail1 = _tc_call(_tail1_body, jax.ShapeDtypeStruct((B * K // 2, LN), F32))
_tail2 = _tc_call(_tail2_body, jax.ShapeDtypeStruct((B, 1), F32))


def kernel(x, edge_index, batch, W0, b0, W1, b1, W2, b2, W3, b3,
           conv1_w, conv1_b, conv2_w, conv2_b,
           lin1_w, lin1_b, lin2_w, lin2_b):
    src3 = edge_index[0].reshape(NW, CH, CW)
    dst3 = edge_index[1].reshape(NW, CH, CW)
    dstf = edge_index[1]
    batch_pad = jnp.concatenate(
        [batch, jnp.full((NW * NBW * CW - N,), B, I32)]).reshape(NW, NBW, CW)

    degp, cntp = _hist(dst3, batch_pad)
    p1, dis, disq, dege, counts, starts = _prep(degp, cntp, x, W0)

    dege_pad = jnp.concatenate([dege.reshape(N), jnp.zeros((SNP - N,), F32)])
    startsN = _ncum(dege_pad)

    hw = _ehist(dstf)
    off = _offs(hw, startsN[:N].reshape(1, N))
    srcS, dstS, normS = _esort(src3, dst3, off, dis.reshape(N))
    srcS3 = srcS.reshape(NW, CH, CW)
    norm2 = normS.reshape(EP, 1)

    def mp(p, hc):
        eg = _egather32 if hc == H else _egather16
        em = _emul32 if hc == H else _emul16
        er = _ereduce32 if hc == H else _ereduce16
        pg = eg(p, srcS3)
        msg = em(pg, norm2)
        a = er(msg.reshape(EP * hc), dstS, startsN)
        return a.reshape(NPADN, hc)

    a1 = mp(p1, H)
    h1, p2 = _mid(a1, p1, disq, b0.reshape(1, H), W1)
    a2 = mp(p2, H)
    h2, p3 = _mid(a2, p2, disq, b1.reshape(1, H), W2)
    a3 = mp(p3, H)
    h3, p4 = _mid_pad(a3, p3, disq, b2.reshape(1, H), W3)
    a4 = mp(p4, LN)
    batchf = batch.astype(F32)[:, None]
    h4, key = _final(a4, p4, disq, b3.reshape(1, 1), batchf)

    xcat = jnp.concatenate(
        [h1, h2, h3, h4, jnp.zeros((N, DP - 97), F32)], axis=1)
    xcat = jnp.concatenate([xcat, jnp.zeros((LN, DP), F32)], axis=0)

    pooled = _select(key.reshape(N), starts.reshape(B), counts.reshape(B),
                     xcat)

    c1w = jnp.pad(conv1_w.reshape(16, 97).T, ((0, DP - 97), (0, 0)))
    c1w2 = jnp.kron(jnp.eye(2, dtype=F32), c1w)
    c1b2 = jnp.tile(conv1_b, 2).reshape(1, 32)
    zp = _tail1(pooled.reshape(B * K // 2, 2 * DP), c1w2, c1b2)

    w2big = sum(
        jnp.pad(jnp.kron(jnp.eye(28, dtype=F32), conv2_w[:, :, j].T),
                ((16 * j, 64 - 16 * j), (0, 0)))
        for j in range(5))
    b2big = jnp.tile(conv2_b, 28).reshape(1, 896)
    lin1p = lin1_w.reshape(32, 28, 128).transpose(1, 0, 2).reshape(896, 128)
    out = _tail2(zp.reshape(B, (K // 2) * LN), w2big, b2big,
                 lin1p, lin1_b.reshape(1, 128), lin2_w, lin2_b.reshape(1, 1))
    return out
